# Initial kernel scaffold; baseline (speedup 1.0000x reference)
import jax
import jax.numpy as jnp
from jax.experimental import pallas as pl


def kernel(x, pos, W, b):
    raise NotImplementedError("write your pallas kernel here")


# speedup vs baseline: 3.6865x; 3.6865x over previous
import functools

import jax
import jax.numpy as jnp
from jax.experimental import pallas as pl

B, C, H, W_IMG = 16, 3, 512, 512
NUM_CLASSES, TOPK_ANCH, KEEP = 21, 200, 100
HW = H * W_IMG
ROWS = B * C
RG = 8
NSEL = (NUM_CLASSES - 1) * TOPK_ANCH


def _feat_body(x_ref, p_ref, o_ref):
    o_ref[...] = jax.lax.dot_general(
        x_ref[...], p_ref[...],
        dimension_numbers=(((1,), (1,)), ((), ())),
        preferred_element_type=jnp.float32)


def _feat_pallas(x2, P):
    return pl.pallas_call(
        _feat_body,
        grid=(ROWS // RG,),
        in_specs=[
            pl.BlockSpec((RG, HW), lambda i: (i, 0)),
            pl.BlockSpec((2, HW), lambda i: (0, 0)),
        ],
        out_specs=pl.BlockSpec((RG, 2), lambda i: (i, 0)),
        out_shape=jax.ShapeDtypeStruct((ROWS, 2), jnp.float32),
    )(x2, P)


def _head_body(f1_ref, f2_ref, wc_ref, bc_ref, wb_ref, bb_ref,
               conf_ref, box_ref):
    f1 = f1_ref[...]
    f2 = f2_ref[...]

    def combo(w, bias):
        a1 = jax.lax.dot_general(f1, w, (((1,), (0,)), ((), ())),
                                 preferred_element_type=jnp.float32) + bias
        a2 = jax.lax.dot_general(f2, w, (((1,), (0,)), ((), ())),
                                 preferred_element_type=jnp.float32) + bias
        return 0.5 * (jnp.tanh(a1) + jnp.tanh(a2))

    conf_ref[...] = jax.nn.sigmoid(combo(wc_ref[...], bc_ref[...]))
    for j in range(4):
        box_ref[j] = combo(wb_ref[3 * j:3 * j + 3], bb_ref[j:j + 1]) * 512.0


def _head_pallas(f1, f2, Wc, bc, Wb, bb):
    return pl.pallas_call(
        _head_body,
        out_shape=(
            jax.ShapeDtypeStruct((B, NSEL), jnp.float32),
            jax.ShapeDtypeStruct((4, B, NSEL), jnp.float32),
        ),
    )(f1, f2, Wc, bc, Wb, bb)


def _topk_body(conf_ref, boxp_ref, sc_ref, lb_ref, bs_ref):
    conf0 = conf_ref[...]
    bp = [boxp_ref[j] for j in range(4)]
    iota_l = jax.lax.broadcasted_iota(jnp.int32, (B, NSEL), 1)
    slot_iota = jax.lax.broadcasted_iota(jnp.int32, (B, 128), 1)
    zf = jnp.zeros((B, 128), jnp.float32)
    zi = jnp.zeros((B, 128), jnp.int32)

    def step(k, carry):
        conf, sc, ix, b0, b1, b2, b3 = carry
        m = jnp.max(conf, axis=1, keepdims=True)
        sel = jnp.min(jnp.where(conf == m, iota_l, NSEL),
                      axis=1, keepdims=True)
        selmask = iota_l == sel
        oh = slot_iota == k
        sc = jnp.where(oh, m, sc)
        ix = jnp.where(oh, sel, ix)
        gath = [jnp.sum(jnp.where(selmask, p, 0.0), axis=1, keepdims=True)
                for p in bp]
        b0 = jnp.where(oh, gath[0], b0)
        b1 = jnp.where(oh, gath[1], b1)
        b2 = jnp.where(oh, gath[2], b2)
        b3 = jnp.where(oh, gath[3], b3)
        conf = jnp.where(selmask, -1.0, conf)
        return conf, sc, ix, b0, b1, b2, b3

    _, sc, ix, b0, b1, b2, b3 = jax.lax.fori_loop(
        0, KEEP, step, (conf0, zf, zi, zf, zf, zf, zf))
    sc_ref[...] = sc[:, :KEEP]
    lb_ref[...] = ix[:, :KEEP] // TOPK_ANCH
    bs_ref[0] = b0[:, :KEEP]
    bs_ref[1] = b1[:, :KEEP]
    bs_ref[2] = b2[:, :KEEP]
    bs_ref[3] = b3[:, :KEEP]


def _topk_pallas(conf, boxp):
    return pl.pallas_call(
        _topk_body,
        out_shape=(
            jax.ShapeDtypeStruct((B, KEEP), jnp.float32),
            jax.ShapeDtypeStruct((B, KEEP), jnp.int32),
            jax.ShapeDtypeStruct((4, B, KEEP), jnp.float32),
        ),
    )(conf, boxp)


def kernel(x, pos, W, b):
    p1 = pos.reshape(HW)
    p2 = pos[::-1, :].reshape(HW)
    P = jnp.stack([p1, p2], axis=0)
    x2 = x.reshape(ROWS, HW)
    fp = _feat_pallas(x2, P)
    f1 = fp[:, 0].reshape(B, C)
    f2 = fp[:, 1].reshape(B, C)

    Wt = W.reshape(C, NUM_CLASSES, TOPK_ANCH, 5)
    bt = b.reshape(NUM_CLASSES, TOPK_ANCH, 5)
    Wc = Wt[:, 1:, :, 0].reshape(C, NSEL)
    bc = bt[1:, :, 0].reshape(1, NSEL)
    Wb = jnp.transpose(Wt[:, 1:, :, 1:], (3, 0, 1, 2)).reshape(4 * C, NSEL)
    bb = jnp.transpose(bt[1:, :, 1:], (2, 0, 1)).reshape(4, NSEL)

    conf, boxp = _head_pallas(f1, f2, Wc, bc, Wb, bb)
    top_scores, labels, bsel = _topk_pallas(conf, boxp)
    sel_boxes = jnp.transpose(bsel, (1, 2, 0))
    return sel_boxes, top_scores, labels


# speedup vs baseline: 4.6780x; 1.2690x over previous
import functools

import jax
import jax.numpy as jnp
from jax.experimental import pallas as pl

B, C, H, W_IMG = 16, 3, 512, 512
NUM_CLASSES, TOPK_ANCH, KEEP = 21, 200, 100
HW = H * W_IMG
ROWS = B * C
RG = 8
NSEL = (NUM_CLASSES - 1) * TOPK_ANCH


def _feat_body(x_ref, p_ref, o_ref):
    xb = x_ref[...].reshape(RG, HW)
    o_ref[...] = jax.lax.dot_general(
        xb, p_ref[...],
        dimension_numbers=(((1,), (1,)), ((), ())),
        preferred_element_type=jnp.float32)


def _feat_pallas(x3, P):
    return pl.pallas_call(
        _feat_body,
        grid=(ROWS // RG,),
        in_specs=[
            pl.BlockSpec((RG, H, W_IMG), lambda i: (i, 0, 0)),
            pl.BlockSpec((2, HW), lambda i: (0, 0)),
        ],
        out_specs=pl.BlockSpec((RG, 2), lambda i: (i, 0)),
        out_shape=jax.ShapeDtypeStruct((ROWS, 2), jnp.float32),
    )(x3, P)


def _head_body(f1_ref, f2_ref, wc_ref, bc_ref, wb_ref, bb_ref,
               conf_ref, box_ref):
    f1 = f1_ref[...]
    f2 = f2_ref[...]

    def combo(w, bias):
        a1 = jax.lax.dot_general(f1, w, (((1,), (0,)), ((), ())),
                                 preferred_element_type=jnp.float32) + bias
        a2 = jax.lax.dot_general(f2, w, (((1,), (0,)), ((), ())),
                                 preferred_element_type=jnp.float32) + bias
        return 0.5 * (jnp.tanh(a1) + jnp.tanh(a2))

    conf_ref[...] = jax.nn.sigmoid(combo(wc_ref[...], bc_ref[...]))
    for j in range(4):
        box_ref[j] = combo(wb_ref[3 * j:3 * j + 3], bb_ref[j:j + 1]) * 512.0


def _head_pallas(f1, f2, Wc, bc, Wb, bb):
    return pl.pallas_call(
        _head_body,
        out_shape=(
            jax.ShapeDtypeStruct((B, NSEL), jnp.float32),
            jax.ShapeDtypeStruct((4, B, NSEL), jnp.float32),
        ),
    )(f1, f2, Wc, bc, Wb, bb)


def _topk_body(conf_ref, boxp_ref, sc_ref, lb_ref, bs_ref):
    conf0 = conf_ref[...]
    bp = [boxp_ref[j] for j in range(4)]
    iota_l = jax.lax.broadcasted_iota(jnp.int32, (B, NSEL), 1)
    slot_iota = jax.lax.broadcasted_iota(jnp.int32, (B, 128), 1)
    zf = jnp.zeros((B, 128), jnp.float32)
    zi = jnp.zeros((B, 128), jnp.int32)

    def step(k, carry):
        conf, sc, ix, b0, b1, b2, b3 = carry
        m = jnp.max(conf, axis=1, keepdims=True)
        sel = jnp.min(jnp.where(conf == m, iota_l, NSEL),
                      axis=1, keepdims=True)
        selmask = iota_l == sel
        oh = slot_iota == k
        sc = jnp.where(oh, m, sc)
        ix = jnp.where(oh, sel, ix)
        gath = [jnp.sum(jnp.where(selmask, p, 0.0), axis=1, keepdims=True)
                for p in bp]
        b0 = jnp.where(oh, gath[0], b0)
        b1 = jnp.where(oh, gath[1], b1)
        b2 = jnp.where(oh, gath[2], b2)
        b3 = jnp.where(oh, gath[3], b3)
        conf = jnp.where(selmask, -1.0, conf)
        return conf, sc, ix, b0, b1, b2, b3

    _, sc, ix, b0, b1, b2, b3 = jax.lax.fori_loop(
        0, KEEP, step, (conf0, zf, zi, zf, zf, zf, zf))
    sc_ref[...] = sc[:, :KEEP]
    lb_ref[...] = ix[:, :KEEP] // TOPK_ANCH
    bs_ref[0] = b0[:, :KEEP]
    bs_ref[1] = b1[:, :KEEP]
    bs_ref[2] = b2[:, :KEEP]
    bs_ref[3] = b3[:, :KEEP]


def _topk_pallas(conf, boxp):
    return pl.pallas_call(
        _topk_body,
        out_shape=(
            jax.ShapeDtypeStruct((B, KEEP), jnp.float32),
            jax.ShapeDtypeStruct((B, KEEP), jnp.int32),
            jax.ShapeDtypeStruct((4, B, KEEP), jnp.float32),
        ),
    )(conf, boxp)


def kernel(x, pos, W, b):
    p1 = pos.reshape(HW)
    p2 = pos[::-1, :].reshape(HW)
    P = jnp.stack([p1, p2], axis=0)
    x3 = x.reshape(ROWS, H, W_IMG)
    fp = _feat_pallas(x3, P)
    f1 = fp[:, 0].reshape(B, C)
    f2 = fp[:, 1].reshape(B, C)

    Wt = W.reshape(C, NUM_CLASSES, TOPK_ANCH, 5)
    bt = b.reshape(NUM_CLASSES, TOPK_ANCH, 5)
    Wc = Wt[:, 1:, :, 0].reshape(C, NSEL)
    bc = bt[1:, :, 0].reshape(1, NSEL)
    Wb = jnp.transpose(Wt[:, 1:, :, 1:], (3, 0, 1, 2)).reshape(4 * C, NSEL)
    bb = jnp.transpose(bt[1:, :, 1:], (2, 0, 1)).reshape(4, NSEL)

    conf, boxp = _head_pallas(f1, f2, Wc, bc, Wb, bb)
    top_scores, labels, bsel = _topk_pallas(conf, boxp)
    sel_boxes = jnp.transpose(bsel, (1, 2, 0))
    return sel_boxes, top_scores, labels


# speedup vs baseline: 4.7650x; 1.0186x over previous
import jax
import jax.numpy as jnp
from jax.experimental import pallas as pl

B, C, H, W_IMG = 16, 3, 512, 512
NUM_CLASSES, TOPK_ANCH, KEEP = 21, 200, 100
HW = H * W_IMG
ROWS = B * C
RG = 8
NSEL = (NUM_CLASSES - 1) * TOPK_ANCH


def _feat_body(x_ref, p_ref, o_ref):
    xb = x_ref[...].reshape(RG, HW)
    o_ref[...] = jax.lax.dot_general(
        xb, p_ref[...],
        dimension_numbers=(((1,), (1,)), ((), ())),
        preferred_element_type=jnp.float32)


def _feat_pallas(x3, P):
    return pl.pallas_call(
        _feat_body,
        grid=(ROWS // RG,),
        in_specs=[
            pl.BlockSpec((RG, H, W_IMG), lambda i: (i, 0, 0)),
            pl.BlockSpec((2, HW), lambda i: (0, 0)),
        ],
        out_specs=pl.BlockSpec((RG, 2), lambda i: (i, 0)),
        out_shape=jax.ShapeDtypeStruct((ROWS, 2), jnp.float32),
    )(x3, P)


def _headtopk_body(f1_ref, f2_ref, wc_ref, bc_ref, wb_ref, bb_ref,
                   sc_ref, lb_ref, bs_ref):
    f1 = f1_ref[...]
    f2 = f2_ref[...]

    def combo(w, bias):
        a1 = jax.lax.dot_general(f1, w, (((1,), (0,)), ((), ())),
                                 preferred_element_type=jnp.float32) + bias
        a2 = jax.lax.dot_general(f2, w, (((1,), (0,)), ((), ())),
                                 preferred_element_type=jnp.float32) + bias
        return 0.5 * (jnp.tanh(a1) + jnp.tanh(a2))

    conf0 = jax.nn.sigmoid(combo(wc_ref[...], bc_ref[...]))
    bp = [combo(wb_ref[3 * j:3 * j + 3], bb_ref[j:j + 1]) * 512.0
          for j in range(4)]

    iota_l = jax.lax.broadcasted_iota(jnp.int32, (B, NSEL), 1)
    slot_iota = jax.lax.broadcasted_iota(jnp.int32, (B, 128), 1)
    zf = jnp.zeros((B, 128), jnp.float32)
    zi = jnp.zeros((B, 128), jnp.int32)

    def step(k, carry):
        conf, sc, ix, b0, b1, b2, b3 = carry
        m = jnp.max(conf, axis=1, keepdims=True)
        sel = jnp.min(jnp.where(conf == m, iota_l, NSEL),
                      axis=1, keepdims=True)
        selmask = iota_l == sel
        oh = slot_iota == k
        sc = jnp.where(oh, m, sc)
        ix = jnp.where(oh, sel, ix)
        gath = [jnp.sum(jnp.where(selmask, p, 0.0), axis=1, keepdims=True)
                for p in bp]
        b0 = jnp.where(oh, gath[0], b0)
        b1 = jnp.where(oh, gath[1], b1)
        b2 = jnp.where(oh, gath[2], b2)
        b3 = jnp.where(oh, gath[3], b3)
        conf = jnp.where(selmask, -1.0, conf)
        return conf, sc, ix, b0, b1, b2, b3

    _, sc, ix, b0, b1, b2, b3 = jax.lax.fori_loop(
        0, KEEP, step, (conf0, zf, zi, zf, zf, zf, zf))
    sc_ref[...] = sc[:, :KEEP]
    lb_ref[...] = ix[:, :KEEP] // TOPK_ANCH
    bs_ref[0] = b0[:, :KEEP]
    bs_ref[1] = b1[:, :KEEP]
    bs_ref[2] = b2[:, :KEEP]
    bs_ref[3] = b3[:, :KEEP]


def _headtopk_pallas(f1, f2, Wc, bc, Wb, bb):
    return pl.pallas_call(
        _headtopk_body,
        out_shape=(
            jax.ShapeDtypeStruct((B, KEEP), jnp.float32),
            jax.ShapeDtypeStruct((B, KEEP), jnp.int32),
            jax.ShapeDtypeStruct((4, B, KEEP), jnp.float32),
        ),
    )(f1, f2, Wc, bc, Wb, bb)


def kernel(x, pos, W, b):
    p1 = pos.reshape(HW)
    p2 = pos[::-1, :].reshape(HW)
    P = jnp.stack([p1, p2], axis=0)
    x3 = x.reshape(ROWS, H, W_IMG)
    fp = _feat_pallas(x3, P)
    f1 = fp[:, 0].reshape(B, C)
    f2 = fp[:, 1].reshape(B, C)

    Wt = W.reshape(C, NUM_CLASSES, TOPK_ANCH, 5)
    bt = b.reshape(NUM_CLASSES, TOPK_ANCH, 5)
    Wc = Wt[:, 1:, :, 0].reshape(C, NSEL)
    bc = bt[1:, :, 0].reshape(1, NSEL)
    Wb = jnp.transpose(Wt[:, 1:, :, 1:], (3, 0, 1, 2)).reshape(4 * C, NSEL)
    bb = jnp.transpose(bt[1:, :, 1:], (2, 0, 1)).reshape(4, NSEL)

    top_scores, labels, bsel = _headtopk_pallas(f1, f2, Wc, bc, Wb, bb)
    sel_boxes = jnp.transpose(bsel, (1, 2, 0))
    return sel_boxes, top_scores, labels
